# baseline (speedup 1.0000x reference)
import jax
import jax.numpy as jnp
from jax import lax
from jax.experimental import pallas as pl
from jax.experimental.pallas import tpu as pltpu
from jax.experimental.pallas import tpu_sc as plsc

_S = 4096
_NW = 32
_RPW = _S // _NW
_L = 16
_NC = _S // _L
_INF = float("inf")


def _sc_body(xs_hbm, y_hbm, rmin_hbm, cmin_hbm,
             xs_v, y0_v, y1_v, y2_v, cm_v, rm_v):
    wid = lax.axis_index("s") * 2 + lax.axis_index("c")
    base = wid * _RPW
    pltpu.sync_copy(xs_hbm.at[pl.ds(base * 3 * _L, _RPW * 3 * _L)], xs_v)
    pltpu.sync_copy(y_hbm.at[pl.ds(0, _S)], y0_v)
    pltpu.sync_copy(y_hbm.at[pl.ds(_S, _S)], y1_v)
    pltpu.sync_copy(y_hbm.at[pl.ds(2 * _S, _S)], y2_v)

    def cm_init(c, _):
        cm_v[pl.ds(c * _L, _L)] = jnp.full((_L,), _INF, jnp.float32)
        return 0

    lax.fori_loop(0, _NC, cm_init, 0, unroll=8)

    def row_quad(q, _):
        i0 = q * 4
        xs = [xs_v[pl.ds((3 * (i0 + k) + c) * _L, _L)]
              for k in range(4) for c in range(3)]
        inf16 = jnp.full((_L,), _INF, jnp.float32)

        def chunk(c, carry):
            r0, r1, r2, r3 = carry
            o = c * _L
            y0 = y0_v[pl.ds(o, _L)]
            y1 = y1_v[pl.ds(o, _L)]
            y2 = y2_v[pl.ds(o, _L)]
            ds = []
            for k in range(4):
                dx = y0 - xs[3 * k]
                dy = y1 - xs[3 * k + 1]
                dz = y2 - xs[3 * k + 2]
                ds.append(dx * dx + dy * dy + dz * dz)
            r0 = jnp.minimum(r0, ds[0])
            r1 = jnp.minimum(r1, ds[1])
            r2 = jnp.minimum(r2, ds[2])
            r3 = jnp.minimum(r3, ds[3])
            m = jnp.minimum(jnp.minimum(ds[0], ds[1]),
                            jnp.minimum(ds[2], ds[3]))
            cm_v[pl.ds(o, _L)] = jnp.minimum(cm_v[pl.ds(o, _L)], m)
            return r0, r1, r2, r3

        r0, r1, r2, r3 = lax.fori_loop(0, _NC, chunk,
                                       (inf16, inf16, inf16, inf16),
                                       unroll=2)
        rm_v[pl.ds((i0 + 0) * _L, _L)] = r0
        rm_v[pl.ds((i0 + 1) * _L, _L)] = r1
        rm_v[pl.ds((i0 + 2) * _L, _L)] = r2
        rm_v[pl.ds((i0 + 3) * _L, _L)] = r3
        return 0

    lax.fori_loop(0, _RPW // 4, row_quad, 0)

    pltpu.sync_copy(rm_v, rmin_hbm.at[pl.ds(base * _L, _RPW * _L)])
    pltpu.sync_copy(cm_v, cmin_hbm.at[wid])


def sc_partials(Xsplat, Yflat):
    mesh = plsc.VectorSubcoreMesh(core_axis_name="c", subcore_axis_name="s")
    kfn = pl.kernel(
        _sc_body,
        mesh=mesh,
        out_type=[
            jax.ShapeDtypeStruct((_S * _L,), jnp.float32),
            jax.ShapeDtypeStruct((_NW, _S), jnp.float32),
        ],
        scratch_types=[
            pltpu.VMEM((_RPW * 3 * _L,), jnp.float32),
            pltpu.VMEM((_S,), jnp.float32),
            pltpu.VMEM((_S,), jnp.float32),
            pltpu.VMEM((_S,), jnp.float32),
            pltpu.VMEM((_S,), jnp.float32),
            pltpu.VMEM((_RPW * _L,), jnp.float32),
        ],
    )
    return kfn(Xsplat, Yflat)


def kernel(X, Y):
    Xc = X[0]
    Yr = jnp.transpose(Y[0], (1, 0))
    Xsplat = jnp.broadcast_to(Xc[:, :, None], (_S, 3, _L)).reshape(-1)
    rmin_p, cmin_p = sc_partials(Xsplat, Yr.reshape(-1))
    row_d2 = jnp.min(rmin_p.reshape(_S, _L), axis=1)
    loss1 = jnp.mean(jnp.sqrt(row_d2))
    col_d2 = jnp.min(cmin_p, axis=0)
    loss2 = jnp.mean(jnp.sqrt(col_d2))
    return loss1 + loss2
